# Initial kernel scaffold; baseline (speedup 1.0000x reference)
import jax
import jax.numpy as jnp
from jax.experimental import pallas as pl


def kernel(x, emb, W1, b1, W2, b2):
    raise NotImplementedError("write your pallas kernel here")


# speedup vs baseline: 7.8954x; 7.8954x over previous
import functools

import jax
import jax.numpy as jnp
from jax import lax
from jax.experimental import pallas as pl
from jax.experimental.pallas import tpu as pltpu
from jax.experimental.pallas import tpu_sc as plsc

SEQ = 50
VOCAB = 1000
EMB = 64
HID = 128
OUT = 1000
BATCH = 16384

NC = 2
NS = 16
NW = NC * NS
BPW = BATCH // NW
CW = 128
CHUNKS = BPW // CW


def _precompute_tables(emb, W1):
    def body(emb_ref, w1_ref, p_ref):
        p_ref[0] = jnp.dot(emb_ref[...], w1_ref[0],
                           preferred_element_type=jnp.float32)

    return pl.pallas_call(
        body,
        grid=(SEQ,),
        in_specs=[
            pl.BlockSpec((VOCAB, EMB), lambda s: (0, 0)),
            pl.BlockSpec((1, EMB, HID), lambda s: (s, 0, 0)),
        ],
        out_specs=pl.BlockSpec((1, VOCAB, HID), lambda s: (s, 0, 0)),
        out_shape=jax.ShapeDtypeStruct((SEQ, VOCAB, HID), jnp.float32),
    )(emb, W1.reshape(SEQ, EMB, HID))


def _sc_bag_sum(p_flat, idx, didx):
    mesh = plsc.VectorSubcoreMesh(core_axis_name="c", subcore_axis_name="s")

    @functools.partial(
        pl.kernel,
        mesh=mesh,
        out_type=jax.ShapeDtypeStruct((BATCH, HID), jnp.float32),
        scratch_types=[
            pltpu.VMEM((SEQ, CHUNKS, CW), jnp.int32),
            pltpu.VMEM((CHUNKS, CW), jnp.int32),
            pltpu.VMEM((CW, HID), jnp.float32),
            pltpu.VMEM_SHARED((BATCH // NC, HID), jnp.float32),
            pltpu.SemaphoreType.DMA,
        ],
    )
    def k(p_hbm, idx_hbm, didx_hbm, g_hbm, idx_v, didx_v, stage_v, acc_sh, sem):
        cid = lax.axis_index("c")
        sid = lax.axis_index("s")
        w = cid * NS + sid
        base = w * BPW
        pltpu.sync_copy(idx_hbm.at[w], idx_v)
        pltpu.sync_copy(didx_hbm.at[sid], didx_v)

        for c in range(CHUNKS):
            pltpu.async_copy(p_hbm.at[idx_v.at[0, c]], stage_v, sem).wait()
            pltpu.sync_copy(stage_v, acc_sh.at[didx_v.at[c]])

        def body(s, carry):
            for c in range(CHUNKS):
                pltpu.async_copy(p_hbm.at[idx_v.at[s, c]], stage_v, sem).wait()
                pltpu.sync_copy(stage_v, acc_sh.at[didx_v.at[c]], add=True)
            return carry

        lax.fori_loop(1, SEQ, body, 0)

        for c in range(CHUNKS):
            pltpu.sync_copy(acc_sh.at[pl.ds(sid * BPW + c * CW, CW)], stage_v)
            pltpu.sync_copy(stage_v, g_hbm.at[pl.ds(base + c * CW, CW)])

    return k(p_flat, idx, didx)


def _mlp_out(g, W2, b1, b2):
    BM = 1024

    def body(g_ref, w2_ref, b1_ref, b2_ref, o_ref):
        h = jnp.maximum(g_ref[...] + b1_ref[...], 0.0)
        o_ref[...] = jnp.dot(h, w2_ref[...],
                             preferred_element_type=jnp.float32) + b2_ref[...]

    return pl.pallas_call(
        body,
        grid=(BATCH // BM,),
        in_specs=[
            pl.BlockSpec((BM, HID), lambda i: (i, 0)),
            pl.BlockSpec((HID, OUT), lambda i: (0, 0)),
            pl.BlockSpec((1, HID), lambda i: (0, 0)),
            pl.BlockSpec((1, OUT), lambda i: (0, 0)),
        ],
        out_specs=pl.BlockSpec((BM, OUT), lambda i: (i, 0)),
        out_shape=jax.ShapeDtypeStruct((BATCH, OUT), jnp.float32),
    )(g, W2, b1.reshape(1, HID), b2.reshape(1, OUT))


def kernel(x, emb, W1, b1, W2, b2):
    p = _precompute_tables(emb, W1)
    p_flat = p.reshape(SEQ * VOCAB, HID)

    flat = x.astype(jnp.int32) + (jnp.arange(SEQ, dtype=jnp.int32) * VOCAB)[None, :]
    idx = flat.reshape(NW, CHUNKS, CW, SEQ).transpose(0, 3, 1, 2)

    didx = (
        (jnp.arange(NS, dtype=jnp.int32) * BPW)[:, None, None]
        + (jnp.arange(CHUNKS, dtype=jnp.int32) * CW)[None, :, None]
        + jnp.arange(CW, dtype=jnp.int32)[None, None, :]
    )

    g = _sc_bag_sum(p_flat, idx, didx)
    return _mlp_out(g, W2, b1, b2)


# speedup vs baseline: 9.1632x; 1.1606x over previous
import functools

import jax
import jax.numpy as jnp
from jax import lax
from jax.experimental import pallas as pl
from jax.experimental.pallas import tpu as pltpu
from jax.experimental.pallas import tpu_sc as plsc

SEQ = 50
VOCAB = 1000
EMB = 64
HID = 128
OUT = 1000
BATCH = 16384

NC = 2
NS = 16
NW = NC * NS
BPW = BATCH // NW
CW = 128
CHUNKS = BPW // CW


def _precompute_tables(emb, W1):
    def body(emb_ref, w1_ref, p_ref):
        p_ref[0] = jnp.dot(emb_ref[...], w1_ref[0],
                           preferred_element_type=jnp.float32)

    return pl.pallas_call(
        body,
        grid=(SEQ,),
        in_specs=[
            pl.BlockSpec((VOCAB, EMB), lambda s: (0, 0)),
            pl.BlockSpec((1, EMB, HID), lambda s: (s, 0, 0)),
        ],
        out_specs=pl.BlockSpec((1, VOCAB, HID), lambda s: (s, 0, 0)),
        out_shape=jax.ShapeDtypeStruct((SEQ, VOCAB, HID), jnp.float32),
    )(emb, W1.reshape(SEQ, EMB, HID))


def _sc_bag_sum(p_flat, idx, didx):
    T = SEQ * CHUNKS
    mesh = plsc.VectorSubcoreMesh(core_axis_name="c", subcore_axis_name="s")

    @functools.partial(
        pl.kernel,
        mesh=mesh,
        out_type=jax.ShapeDtypeStruct((BATCH, HID), jnp.float32),
        scratch_types=[
            pltpu.VMEM((T, CW), jnp.int32),
            pltpu.VMEM((CHUNKS, CW), jnp.int32),
            pltpu.VMEM((2, CW, HID), jnp.float32),
            pltpu.VMEM_SHARED((BATCH // NC, HID), jnp.float32),
            pltpu.SemaphoreType.DMA,
            pltpu.SemaphoreType.DMA,
            pltpu.SemaphoreType.DMA,
            pltpu.SemaphoreType.DMA,
        ],
    )
    def k(p_hbm, idx_hbm, didx_hbm, g_hbm, idx_v, didx_v, stage_v, acc_sh,
          gsem0, gsem1, ssem0, ssem1):
        cid = lax.axis_index("c")
        sid = lax.axis_index("s")
        w = cid * NS + sid
        base = w * BPW
        pltpu.sync_copy(idx_hbm.at[w], idx_v)
        pltpu.sync_copy(didx_hbm.at[sid], didx_v)

        gsems = (gsem0, gsem1)
        ssems = (ssem0, ssem1)

        def fire_gather(t, h):
            pltpu.async_copy(p_hbm.at[idx_v.at[t]], stage_v.at[h], gsems[h])

        def wait_gather(h):
            pltpu.make_async_copy(p_hbm.at[pl.ds(0, CW)], stage_v.at[h],
                                  gsems[h]).wait()

        def fire_scatter(t, h):
            pltpu.async_copy(stage_v.at[h], acc_sh.at[didx_v.at[t % CHUNKS]],
                             ssems[h], add=True)

        def wait_scatter(h):
            pltpu.make_async_copy(stage_v.at[h], acc_sh.at[pl.ds(0, CW)],
                                  ssems[h]).wait()

        for c in range(CHUNKS):
            pltpu.async_copy(p_hbm.at[idx_v.at[c]], stage_v.at[0],
                             gsem0).wait()
            pltpu.sync_copy(stage_v.at[0], acc_sh.at[didx_v.at[c]])

        fire_gather(CHUNKS, 0)
        fire_gather(CHUNKS + 1, 1)

        def body(g, carry):
            t0 = CHUNKS + 2 * g
            for h, t in ((0, t0), (1, t0 + 1)):
                wait_gather(h)
                fire_scatter(t, h)
            for h, t in ((0, t0), (1, t0 + 1)):
                wait_scatter(h)

                @pl.when(t + 2 < T)
                def _():
                    fire_gather(t + 2, h)

            return carry

        lax.fori_loop(0, (T - CHUNKS) // 2, body, 0)

        for c in range(CHUNKS):
            pltpu.sync_copy(acc_sh.at[pl.ds(sid * BPW + c * CW, CW)],
                            stage_v.at[0])
            pltpu.sync_copy(stage_v.at[0], g_hbm.at[pl.ds(base + c * CW, CW)])

    return k(p_flat, idx, didx)


def _mlp_out(g, W2, b1, b2):
    BM = 1024

    def body(g_ref, w2_ref, b1_ref, b2_ref, o_ref):
        h = jnp.maximum(g_ref[...] + b1_ref[...], 0.0)
        o_ref[...] = jnp.dot(h, w2_ref[...],
                             preferred_element_type=jnp.float32) + b2_ref[...]

    return pl.pallas_call(
        body,
        grid=(BATCH // BM,),
        in_specs=[
            pl.BlockSpec((BM, HID), lambda i: (i, 0)),
            pl.BlockSpec((HID, OUT), lambda i: (0, 0)),
            pl.BlockSpec((1, HID), lambda i: (0, 0)),
            pl.BlockSpec((1, OUT), lambda i: (0, 0)),
        ],
        out_specs=pl.BlockSpec((BM, OUT), lambda i: (i, 0)),
        out_shape=jax.ShapeDtypeStruct((BATCH, OUT), jnp.float32),
    )(g, W2, b1.reshape(1, HID), b2.reshape(1, OUT))


def kernel(x, emb, W1, b1, W2, b2):
    p = _precompute_tables(emb, W1)
    p_flat = p.reshape(SEQ * VOCAB, HID)

    flat = x.astype(jnp.int32) + (jnp.arange(SEQ, dtype=jnp.int32) * VOCAB)[None, :]
    idx = (flat.reshape(NW, CHUNKS, CW, SEQ).transpose(0, 3, 1, 2)
           .reshape(NW, SEQ * CHUNKS, CW))

    didx = (
        (jnp.arange(NS, dtype=jnp.int32) * BPW)[:, None, None]
        + (jnp.arange(CHUNKS, dtype=jnp.int32) * CW)[None, :, None]
        + jnp.arange(CW, dtype=jnp.int32)[None, None, :]
    )

    g = _sc_bag_sum(p_flat, idx, didx)
    return _mlp_out(g, W2, b1, b2)


# speedup vs baseline: 10.8158x; 1.1803x over previous
import functools

import jax
import jax.numpy as jnp
from jax import lax
from jax.experimental import pallas as pl
from jax.experimental.pallas import tpu as pltpu
from jax.experimental.pallas import tpu_sc as plsc

SEQ = 50
VOCAB = 1000
EMB = 64
HID = 128
OUT = 1000
BATCH = 16384

NC = 2
NS = 16
NW = NC * NS
HALVES = 2
BH = BATCH // HALVES
BPW = BH // NW
CW = 128
T = BPW * SEQ // CW
NBUF = 4
RB = BPW // CW


def _precompute_tables(emb, W1):
    def body(emb_ref, w1_ref, p_ref):
        p_ref[0] = jnp.dot(emb_ref[...], w1_ref[0],
                           preferred_element_type=jnp.float32)

    return pl.pallas_call(
        body,
        grid=(SEQ,),
        in_specs=[
            pl.BlockSpec((VOCAB, EMB), lambda s: (0, 0)),
            pl.BlockSpec((1, EMB, HID), lambda s: (s, 0, 0)),
        ],
        out_specs=pl.BlockSpec((1, VOCAB, HID), lambda s: (s, 0, 0)),
        out_shape=jax.ShapeDtypeStruct((SEQ, VOCAB, HID), jnp.float32),
    )(emb, W1.reshape(SEQ, EMB, HID))


def _sc_bag_sum(p_flat, idx, didx):
    mesh = plsc.VectorSubcoreMesh(core_axis_name="c", subcore_axis_name="s")

    @functools.partial(
        pl.kernel,
        mesh=mesh,
        out_type=jax.ShapeDtypeStruct((BH, HID), jnp.float32),
        scratch_types=[
            pltpu.VMEM((T, CW), jnp.int32),
            pltpu.VMEM((T, CW), jnp.int32),
            pltpu.VMEM((NBUF, CW, HID), jnp.float32),
            pltpu.VMEM_SHARED((BH // NC, HID), jnp.float32),
            pltpu.SemaphoreType.DMA, pltpu.SemaphoreType.DMA,
            pltpu.SemaphoreType.DMA, pltpu.SemaphoreType.DMA,
            pltpu.SemaphoreType.DMA, pltpu.SemaphoreType.DMA,
            pltpu.SemaphoreType.DMA, pltpu.SemaphoreType.DMA,
        ],
    )
    def k(p_hbm, idx_hbm, didx_hbm, g_hbm, idx_v, didx_v, stage_v, acc_sh,
          g0, g1, g2, g3, s0, s1, s2, s3):
        cid = lax.axis_index("c")
        sid = lax.axis_index("s")
        w = cid * NS + sid
        base = w * BPW
        gsems = (g0, g1, g2, g3)
        ssems = (s0, s1, s2, s3)

        pltpu.sync_copy(idx_hbm.at[w], idx_v)
        pltpu.sync_copy(didx_hbm.at[sid], didx_v)

        def zbody(r, carry):
            for kk in range(HID // 16):
                stage_v[0, r, pl.ds(kk * 16, 16)] = jnp.zeros((16,),
                                                              jnp.float32)
            return carry

        lax.fori_loop(0, CW, zbody, 0)
        for c in range(RB):
            pltpu.sync_copy(stage_v.at[0],
                            acc_sh.at[pl.ds(sid * BPW + c * CW, CW)])

        def fire_gather(t, h):
            pltpu.async_copy(p_hbm.at[idx_v.at[t]], stage_v.at[h], gsems[h])

        def wait_gather(h):
            pltpu.make_async_copy(p_hbm.at[pl.ds(0, CW)], stage_v.at[h],
                                  gsems[h]).wait()

        def fire_scatter(t, h):
            pltpu.async_copy(stage_v.at[h], acc_sh.at[didx_v.at[t]],
                             ssems[h], add=True)

        def wait_scatter(h):
            pltpu.make_async_copy(stage_v.at[h], acc_sh.at[pl.ds(0, CW)],
                                  ssems[h]).wait()

        for h in range(NBUF):
            fire_gather(h, h)

        def body(g, carry):
            t0 = NBUF * g
            for h in range(NBUF):
                wait_gather(h)
                fire_scatter(t0 + h, h)
            for h in range(NBUF):
                wait_scatter(h)

                @pl.when(t0 + h + NBUF < T)
                def _():
                    fire_gather(t0 + h + NBUF, h)

            return carry

        lax.fori_loop(0, T // NBUF, body, 0)

        for c in range(RB):
            pltpu.sync_copy(acc_sh.at[pl.ds(sid * BPW + c * CW, CW)],
                            stage_v.at[0])
            pltpu.sync_copy(stage_v.at[0], g_hbm.at[pl.ds(base + c * CW, CW)])

    return k(p_flat, idx, didx)


def _mlp_out(g, W2, b1, b2, rows, row0, out_prev=None):
    BM = 1024
    blk0 = row0 // BM

    def body(g_ref, w2_ref, b1_ref, b2_ref, *rest):
        o_ref = rest[-1]
        h = jnp.maximum(g_ref[...] + b1_ref[...], 0.0)
        o_ref[...] = jnp.dot(h, w2_ref[...],
                             preferred_element_type=jnp.float32) + b2_ref[...]

    in_specs = [
        pl.BlockSpec((BM, HID), lambda i: (i, 0)),
        pl.BlockSpec((HID, OUT), lambda i: (0, 0)),
        pl.BlockSpec((1, HID), lambda i: (0, 0)),
        pl.BlockSpec((1, OUT), lambda i: (0, 0)),
    ]
    args = [g, W2, b1.reshape(1, HID), b2.reshape(1, OUT)]
    kwargs = {}
    if out_prev is not None:
        in_specs.append(pl.BlockSpec(memory_space=pltpu.HBM))
        args.append(out_prev)
        kwargs["input_output_aliases"] = {4: 0}
    return pl.pallas_call(
        body,
        grid=(rows // BM,),
        in_specs=in_specs,
        out_specs=pl.BlockSpec((BM, OUT), lambda i: (i + blk0, 0)),
        out_shape=jax.ShapeDtypeStruct((BATCH, OUT), jnp.float32),
        **kwargs,
    )(*args)


def kernel(x, emb, W1, b1, W2, b2):
    p = _precompute_tables(emb, W1)
    p_flat = p.reshape(SEQ * VOCAB, HID)

    flat = x.astype(jnp.int32) + (jnp.arange(SEQ, dtype=jnp.int32) * VOCAB)[None, :]

    didx = (
        (jnp.arange(NS, dtype=jnp.int32) * BPW)[:, None, None]
        + (jnp.arange(T * CW, dtype=jnp.int32) // SEQ).reshape(T, CW)[None]
    )

    halves = []
    for hf in range(HALVES):
        idx_h = flat[hf * BH:(hf + 1) * BH].reshape(NW, T, CW)
        halves.append(_sc_bag_sum(p_flat, idx_h, didx))

    out = _mlp_out(halves[0], W2, b1, b2, BH, 0)
    out = _mlp_out(halves[1], W2, b1, b2, BH, BH, out_prev=out)
    return out


# speedup vs baseline: 11.1468x; 1.0306x over previous
import functools

import jax
import jax.numpy as jnp
from jax import lax
from jax.experimental import pallas as pl
from jax.experimental.pallas import tpu as pltpu
from jax.experimental.pallas import tpu_sc as plsc

SEQ = 50
VOCAB = 1000
EMB = 64
HID = 128
OUT = 1000
BATCH = 16384

NC = 2
NS = 16
NW = NC * NS
HALVES = 2
BH = BATCH // HALVES
BPW = BH // NW
CW = 128
T = BPW * SEQ // CW
NBUF = 4
RB = BPW // CW


def _precompute_tables(emb, W1):
    SB = 10

    def body(emb_ref, w1_ref, p_ref):
        for i in range(SB):
            p_ref[i] = jnp.dot(emb_ref[...], w1_ref[i],
                               preferred_element_type=jnp.float32)

    return pl.pallas_call(
        body,
        grid=(SEQ // SB,),
        in_specs=[
            pl.BlockSpec((VOCAB, EMB), lambda s: (0, 0)),
            pl.BlockSpec((SB, EMB, HID), lambda s: (s, 0, 0)),
        ],
        out_specs=pl.BlockSpec((SB, VOCAB, HID), lambda s: (s, 0, 0)),
        out_shape=jax.ShapeDtypeStruct((SEQ, VOCAB, HID), jnp.float32),
    )(emb, W1.reshape(SEQ, EMB, HID))


def _sc_bag_sum(p_flat, idx, didx):
    mesh = plsc.VectorSubcoreMesh(core_axis_name="c", subcore_axis_name="s")

    @functools.partial(
        pl.kernel,
        mesh=mesh,
        out_type=jax.ShapeDtypeStruct((BH, HID), jnp.float32),
        scratch_types=[
            pltpu.VMEM((T, CW), jnp.int32),
            pltpu.VMEM((T, CW), jnp.int32),
            pltpu.VMEM((NBUF, CW, HID), jnp.float32),
            pltpu.VMEM_SHARED((BH // NC, HID), jnp.float32),
        ] + [pltpu.SemaphoreType.DMA] * (2 * NBUF),
    )
    def k(p_hbm, idx_hbm, didx_hbm, g_hbm, idx_v, didx_v, stage_v, acc_sh,
          *sems):
        cid = lax.axis_index("c")
        sid = lax.axis_index("s")
        w = cid * NS + sid
        base = w * BPW
        gsems = sems[:NBUF]
        ssems = sems[NBUF:]

        pltpu.sync_copy(idx_hbm.at[w], idx_v)
        pltpu.sync_copy(didx_hbm.at[sid], didx_v)

        def zbody(r, carry):
            for kk in range(HID // 16):
                stage_v[0, r, pl.ds(kk * 16, 16)] = jnp.zeros((16,),
                                                              jnp.float32)
            return carry

        lax.fori_loop(0, CW, zbody, 0)
        for c in range(RB):
            pltpu.sync_copy(stage_v.at[0],
                            acc_sh.at[pl.ds(sid * BPW + c * CW, CW)])

        def fire_gather(t, h):
            pltpu.async_copy(p_hbm.at[idx_v.at[t]], stage_v.at[h], gsems[h])

        def wait_gather(h):
            pltpu.make_async_copy(p_hbm.at[pl.ds(0, CW)], stage_v.at[h],
                                  gsems[h]).wait()

        def fire_scatter(t, h):
            pltpu.async_copy(stage_v.at[h], acc_sh.at[didx_v.at[t]],
                             ssems[h], add=True)

        def wait_scatter(h):
            pltpu.make_async_copy(stage_v.at[h], acc_sh.at[pl.ds(0, CW)],
                                  ssems[h]).wait()

        for h in range(NBUF):
            fire_gather(h, h)

        def body(g, carry):
            t0 = NBUF * g
            for h in range(NBUF):
                wait_gather(h)
                fire_scatter(t0 + h, h)
            for h in range(NBUF):
                wait_scatter(h)

                @pl.when(t0 + h + NBUF < T)
                def _():
                    fire_gather(t0 + h + NBUF, h)

            return carry

        lax.fori_loop(0, T // NBUF, body, 0)

        for c in range(RB):
            pltpu.sync_copy(acc_sh.at[pl.ds(sid * BPW + c * CW, CW)],
                            stage_v.at[0])
            pltpu.sync_copy(stage_v.at[0], g_hbm.at[pl.ds(base + c * CW, CW)])

    return k(p_flat, idx, didx)


def _mlp_out(g0, g1, W2, b1, b2):
    BM = 1024
    NB0 = BH // BM

    def body(g0_ref, g1_ref, w2_ref, b1_ref, b2_ref, o_ref):
        pid = pl.program_id(0)
        g = jnp.where(pid < NB0, g0_ref[...], g1_ref[...])
        h = jnp.maximum(g + b1_ref[...], 0.0)
        o_ref[...] = jnp.dot(h, w2_ref[...],
                             preferred_element_type=jnp.float32) + b2_ref[...]

    return pl.pallas_call(
        body,
        grid=(BATCH // BM,),
        in_specs=[
            pl.BlockSpec((BM, HID), lambda i: (jnp.minimum(i, NB0 - 1), 0)),
            pl.BlockSpec((BM, HID), lambda i: (jnp.maximum(i - NB0, 0), 0)),
            pl.BlockSpec((HID, OUT), lambda i: (0, 0)),
            pl.BlockSpec((1, HID), lambda i: (0, 0)),
            pl.BlockSpec((1, OUT), lambda i: (0, 0)),
        ],
        out_specs=pl.BlockSpec((BM, OUT), lambda i: (i, 0)),
        out_shape=jax.ShapeDtypeStruct((BATCH, OUT), jnp.float32),
    )(g0, g1, W2, b1.reshape(1, HID), b2.reshape(1, OUT))


def kernel(x, emb, W1, b1, W2, b2):
    p = _precompute_tables(emb, W1)
    p_flat = p.reshape(SEQ * VOCAB, HID)

    flat = x.astype(jnp.int32) + (jnp.arange(SEQ, dtype=jnp.int32) * VOCAB)[None, :]

    didx = (
        (jnp.arange(NS, dtype=jnp.int32) * BPW)[:, None, None]
        + (jnp.arange(T * CW, dtype=jnp.int32) // SEQ).reshape(T, CW)[None]
    )

    halves = []
    for hf in range(HALVES):
        idx_h = flat[hf * BH:(hf + 1) * BH].reshape(NW, T, CW)
        halves.append(_sc_bag_sum(p_flat, idx_h, didx))

    return _mlp_out(halves[0], halves[1], W2, b1, b2)


# speedup vs baseline: 13.1076x; 1.1759x over previous
import functools

import jax
import jax.numpy as jnp
from jax import lax
from jax.experimental import pallas as pl
from jax.experimental.pallas import tpu as pltpu
from jax.experimental.pallas import tpu_sc as plsc

SEQ = 50
VOCAB = 1000
EMB = 64
HID = 128
OUT = 1000
BATCH = 16384

NC = 2
NS = 16
NW = NC * NS
HALVES = 2
BH = BATCH // HALVES
BPW = BH // NW
CW = 128
T = BPW * SEQ // CW
NBUF = 4
RB = BPW // CW


def _precompute_tables(emb, W1):
    SB = 10

    def body(emb_ref, w1_ref, p_ref):
        for i in range(SB):
            p_ref[i] = jnp.dot(emb_ref[...], w1_ref[i],
                               preferred_element_type=jnp.float32)

    return pl.pallas_call(
        body,
        grid=(SEQ // SB,),
        in_specs=[
            pl.BlockSpec((VOCAB, EMB), lambda s: (0, 0)),
            pl.BlockSpec((SB, EMB, HID), lambda s: (s, 0, 0)),
        ],
        out_specs=pl.BlockSpec((SB, VOCAB, HID), lambda s: (s, 0, 0)),
        out_shape=jax.ShapeDtypeStruct((SEQ, VOCAB, HID), jnp.float32),
    )(emb, W1.reshape(SEQ, EMB, HID))


def _sc_bag_sum(p_flat, idx, didx):
    mesh = plsc.VectorSubcoreMesh(core_axis_name="c", subcore_axis_name="s")

    @functools.partial(
        pl.kernel,
        mesh=mesh,
        out_type=jax.ShapeDtypeStruct((BH, HID), jnp.float32),
        scratch_types=[
            pltpu.VMEM((T, CW), jnp.int32),
            pltpu.VMEM((T, CW), jnp.int32),
            pltpu.VMEM((NBUF, CW, HID), jnp.float32),
            pltpu.VMEM_SHARED((BH // NC, HID), jnp.float32),
        ] + [pltpu.SemaphoreType.DMA] * (2 * NBUF),
    )
    def k(p_hbm, idx_hbm, didx_hbm, g_hbm, idx_v, didx_v, stage_v, acc_sh,
          *sems):
        cid = lax.axis_index("c")
        sid = lax.axis_index("s")
        w = cid * NS + sid
        base = w * BPW
        gsems = sems[:NBUF]
        ssems = sems[NBUF:]

        pltpu.sync_copy(idx_hbm.at[w], idx_v)
        pltpu.sync_copy(didx_hbm.at[sid], didx_v)

        def zbody(r, carry):
            for kk in range(HID // 16):
                stage_v[0, r, pl.ds(kk * 16, 16)] = jnp.zeros((16,),
                                                              jnp.float32)
            return carry

        lax.fori_loop(0, CW, zbody, 0)
        for c in range(RB):
            pltpu.sync_copy(stage_v.at[0],
                            acc_sh.at[pl.ds(sid * BPW + c * CW, CW)])

        def fire_gather(t, h):
            pltpu.async_copy(p_hbm.at[idx_v.at[t]], stage_v.at[h], gsems[h])

        def wait_gather(h):
            pltpu.make_async_copy(p_hbm.at[pl.ds(0, CW)], stage_v.at[h],
                                  gsems[h]).wait()

        def fire_scatter(t, h):
            pltpu.async_copy(stage_v.at[h], acc_sh.at[didx_v.at[t]],
                             ssems[h], add=True)

        def wait_scatter(h):
            pltpu.make_async_copy(stage_v.at[h], acc_sh.at[pl.ds(0, CW)],
                                  ssems[h]).wait()

        for h in range(NBUF):
            fire_gather(h, h)

        def body(g, carry):
            t0 = NBUF * g
            for h in range(NBUF):
                wait_gather(h)
                fire_scatter(t0 + h, h)
            for h in range(NBUF):
                wait_scatter(h)

                @pl.when(t0 + h + NBUF < T)
                def _():
                    fire_gather(t0 + h + NBUF, h)

            return carry

        lax.fori_loop(0, T // NBUF, body, 0)

        for c in range(RB):
            pltpu.sync_copy(acc_sh.at[pl.ds(sid * BPW + c * CW, CW)],
                            stage_v.at[0])
            pltpu.sync_copy(stage_v.at[0], g_hbm.at[pl.ds(base + c * CW, CW)])

    return k(p_flat, idx, didx)


def _mlp_out(g0, g1, W2, b1, b2):
    BM = 1024
    NB0 = BH // BM

    def body(g0_ref, g1_ref, w2_ref, b1_ref, b2_ref, o_ref):
        pid = pl.program_id(0)
        g = jnp.where(pid < NB0, g0_ref[...], g1_ref[...])
        h = jnp.maximum(g + b1_ref[...], 0.0)
        ot = lax.dot_general(w2_ref[...], h, (((0,), (1,)), ((), ())),
                             preferred_element_type=jnp.float32)
        o_ref[...] = ot + b2_ref[...]

    out_t = pl.pallas_call(
        body,
        grid=(BATCH // BM,),
        in_specs=[
            pl.BlockSpec((BM, HID), lambda i: (jnp.minimum(i, NB0 - 1), 0)),
            pl.BlockSpec((BM, HID), lambda i: (jnp.maximum(i - NB0, 0), 0)),
            pl.BlockSpec((HID, OUT), lambda i: (0, 0)),
            pl.BlockSpec((1, HID), lambda i: (0, 0)),
            pl.BlockSpec((OUT, 1), lambda i: (0, 0)),
        ],
        out_specs=pl.BlockSpec((OUT, BM), lambda i: (0, i)),
        out_shape=jax.ShapeDtypeStruct((OUT, BATCH), jnp.float32),
    )(g0, g1, W2, b1.reshape(1, HID), b2.reshape(OUT, 1))
    return out_t.T


def kernel(x, emb, W1, b1, W2, b2):
    p = _precompute_tables(emb, W1)
    p_flat = p.reshape(SEQ * VOCAB, HID)

    flat = x.astype(jnp.int32) + (jnp.arange(SEQ, dtype=jnp.int32) * VOCAB)[None, :]

    didx = (
        (jnp.arange(NS, dtype=jnp.int32) * BPW)[:, None, None]
        + (jnp.arange(T * CW, dtype=jnp.int32) // SEQ).reshape(T, CW)[None]
    )

    halves = []
    for hf in range(HALVES):
        idx_h = flat[hf * BH:(hf + 1) * BH].reshape(NW, T, CW)
        halves.append(_sc_bag_sum(p_flat, idx_h, didx))

    return _mlp_out(halves[0], halves[1], W2, b1, b2)


# speedup vs baseline: 13.4043x; 1.0226x over previous
import functools

import jax
import jax.numpy as jnp
from jax import lax
from jax.experimental import pallas as pl
from jax.experimental.pallas import tpu as pltpu
from jax.experimental.pallas import tpu_sc as plsc

SEQ = 50
VOCAB = 1000
EMB = 64
HID = 128
OUT = 1000
BATCH = 16384

NC = 2
NS = 16
NW = NC * NS
HALVES = 2
BH = BATCH // HALVES
BPW = BH // NW
CW = 128
T = BPW * SEQ // CW
NBUF = 4
RB = BPW // CW


def _precompute_tables(emb, W1):
    SB = 10

    def body(emb_ref, w1_ref, p_ref):
        for i in range(SB):
            p_ref[i] = jnp.dot(emb_ref[...], w1_ref[i],
                               preferred_element_type=jnp.float32)

    return pl.pallas_call(
        body,
        grid=(SEQ // SB,),
        in_specs=[
            pl.BlockSpec((VOCAB, EMB), lambda s: (0, 0)),
            pl.BlockSpec((SB, EMB, HID), lambda s: (s, 0, 0)),
        ],
        out_specs=pl.BlockSpec((SB, VOCAB, HID), lambda s: (s, 0, 0)),
        out_shape=jax.ShapeDtypeStruct((SEQ, VOCAB, HID), jnp.float32),
    )(emb, W1.reshape(SEQ, EMB, HID))


def _sc_bag_sum(p_flat, idx, didx):
    mesh = plsc.VectorSubcoreMesh(core_axis_name="c", subcore_axis_name="s")

    @functools.partial(
        pl.kernel,
        mesh=mesh,
        out_type=jax.ShapeDtypeStruct((BH, HID), jnp.float32),
        scratch_types=[
            pltpu.VMEM((T, CW), jnp.int32),
            pltpu.VMEM((T, CW), jnp.int32),
            pltpu.VMEM((NBUF, CW, HID), jnp.float32),
            pltpu.VMEM_SHARED((BH // NC, HID), jnp.float32),
        ] + [pltpu.SemaphoreType.DMA] * (2 * NBUF),
    )
    def k(p_hbm, idx_hbm, didx_hbm, g_hbm, idx_v, didx_v, stage_v, acc_sh,
          *sems):
        cid = lax.axis_index("c")
        sid = lax.axis_index("s")
        w = cid * NS + sid
        base = w * BPW
        gsems = sems[:NBUF]
        ssems = sems[NBUF:]

        pltpu.sync_copy(idx_hbm.at[w], idx_v)
        pltpu.sync_copy(didx_hbm.at[sid], didx_v)

        def zbody(r, carry):
            for kk in range(HID // 16):
                stage_v[0, r, pl.ds(kk * 16, 16)] = jnp.zeros((16,),
                                                              jnp.float32)
            return carry

        lax.fori_loop(0, CW, zbody, 0)
        for c in range(RB):
            pltpu.sync_copy(stage_v.at[0],
                            acc_sh.at[pl.ds(sid * BPW + c * CW, CW)])

        def fire_gather(t, h):
            pltpu.async_copy(p_hbm.at[idx_v.at[t]], stage_v.at[h], gsems[h])

        def wait_gather(h):
            pltpu.make_async_copy(p_hbm.at[pl.ds(0, CW)], stage_v.at[h],
                                  gsems[h]).wait()

        def fire_scatter(t, h):
            pltpu.async_copy(stage_v.at[h], acc_sh.at[didx_v.at[t]],
                             ssems[h], add=True)

        def wait_scatter(h):
            pltpu.make_async_copy(stage_v.at[h], acc_sh.at[pl.ds(0, CW)],
                                  ssems[h]).wait()

        for h in range(NBUF):
            fire_gather(h, h)

        def body(g, carry):
            t0 = NBUF * g
            for h in range(NBUF):
                wait_gather(h)
                fire_scatter(t0 + h, h)
            for h in range(NBUF):
                wait_scatter(h)

                @pl.when(t0 + h + NBUF < T)
                def _():
                    fire_gather(t0 + h + NBUF, h)

            return carry

        lax.fori_loop(0, T // NBUF, body, 0)

        for c in range(RB):
            pltpu.sync_copy(acc_sh.at[pl.ds(sid * BPW + c * CW, CW)],
                            stage_v.at[0])
            pltpu.sync_copy(stage_v.at[0], g_hbm.at[pl.ds(base + c * CW, CW)])

    return k(p_flat, idx, didx)


def _mlp_out_half(g, W2, b1, b2, hf, out_prev=None):
    BM = 1024
    blk0 = hf * (BH // BM)

    def body(g_ref, w2_ref, b1_ref, b2_ref, *rest):
        o_ref = rest[-1]
        h = jnp.maximum(g_ref[...] + b1_ref[...], 0.0)
        ot = lax.dot_general(w2_ref[...], h, (((0,), (1,)), ((), ())),
                             preferred_element_type=jnp.float32)
        o_ref[...] = ot + b2_ref[...]

    in_specs = [
        pl.BlockSpec((BM, HID), lambda i: (i, 0)),
        pl.BlockSpec((HID, OUT), lambda i: (0, 0)),
        pl.BlockSpec((1, HID), lambda i: (0, 0)),
        pl.BlockSpec((OUT, 1), lambda i: (0, 0)),
    ]
    args = [g, W2, b1.reshape(1, HID), b2.reshape(OUT, 1)]
    kwargs = {}
    if out_prev is not None:
        in_specs.append(pl.BlockSpec(memory_space=pltpu.HBM))
        args.append(out_prev)
        kwargs["input_output_aliases"] = {4: 0}
    return pl.pallas_call(
        body,
        grid=(BH // BM,),
        in_specs=in_specs,
        out_specs=pl.BlockSpec((OUT, BM), lambda i: (0, i + blk0)),
        out_shape=jax.ShapeDtypeStruct((OUT, BATCH), jnp.float32),
        **kwargs,
    )(*args)


def kernel(x, emb, W1, b1, W2, b2):
    p = _precompute_tables(emb, W1)
    p_flat = p.reshape(SEQ * VOCAB, HID)

    flat = x.astype(jnp.int32) + (jnp.arange(SEQ, dtype=jnp.int32) * VOCAB)[None, :]

    didx = (
        (jnp.arange(NS, dtype=jnp.int32) * BPW)[:, None, None]
        + (jnp.arange(T * CW, dtype=jnp.int32) // SEQ).reshape(T, CW)[None]
    )

    halves = []
    for hf in range(HALVES):
        idx_h = flat[hf * BH:(hf + 1) * BH].reshape(NW, T, CW)
        halves.append(_sc_bag_sum(p_flat, idx_h, didx))

    out_t = _mlp_out_half(halves[0], W2, b1, b2, 0)
    out_t = _mlp_out_half(halves[1], W2, b1, b2, 1, out_prev=out_t)
    return out_t.T
